# baseline (speedup 1.0000x reference)
import jax
import jax.numpy as jnp
from jax.experimental import pallas as pl
from jax.experimental.pallas import tpu as pltpu

N = 50000
H = 64
OUT = 2


def _head_body(h_ref, w_ref, b_ref, o_ref):
    o_ref[...] = h_ref[...] @ w_ref[...] + b_ref[...]


def _layer(h, pos, src, dst, Wa, ba, Wb, bb):
    Wa_h = Wa[: h.shape[1]]
    Wa_p = Wa[h.shape[1] :]
    A = h @ Wa_h + pos @ Wa_p + ba
    B = pos @ Wa_p
    z = A[src] - B[dst]
    m = jnp.maximum(z, 0.0) @ Wb + bb
    m = jnp.maximum(m, 0.0)
    out = jax.ops.segment_max(m, dst, num_segments=N)
    return jnp.where(jnp.isneginf(out), 0.0, out)


def kernel(x, pos, edge_index, W0a, b0a, W0b, b0b, W1a, b1a, W1b, b1b,
           W2a, b2a, W2b, b2b, Wh, bh):
    src = edge_index[0]
    dst = edge_index[1]
    dst, src = jax.lax.sort([dst, src], num_keys=1)
    h = _layer(x, pos, src, dst, W0a, b0a, W0b, b0b)
    h = _layer(h, pos, src, dst, W1a, b1a, W1b, b1b)
    h = _layer(h, pos, src, dst, W2a, b2a, W2b, b2b)
    blk = 2000
    out = pl.pallas_call(
        _head_body,
        grid=(N // blk,),
        in_specs=[
            pl.BlockSpec((blk, H), lambda i: (i, 0)),
            pl.BlockSpec((H, OUT), lambda i: (0, 0)),
            pl.BlockSpec((OUT,), lambda i: (0,)),
        ],
        out_specs=pl.BlockSpec((blk, OUT), lambda i: (i, 0)),
        out_shape=jax.ShapeDtypeStruct((N, OUT), jnp.float32),
    )(h, Wh, bh)
    return out


# speedup vs baseline: 2.6512x; 2.6512x over previous
import functools

import jax
import jax.numpy as jnp
from jax import lax
from jax.experimental import pallas as pl
from jax.experimental.pallas import tpu as pltpu
from jax.experimental.pallas import tpu_sc as plsc

N = 50000
E = 800000
H = 64
OUT = 2

NW = 32
RNG = 1568
NPAD = NW * RNG
KG = 128
NCHG = E // KG
KS = 128
NBUF = 2
EROW = E // 2
EROWP = (E + KS) // 2
NBLK = 2000

_MESH = plsc.VectorSubcoreMesh(
    core_axis_name="c", subcore_axis_name="s", num_cores=2, num_subcores=16)


def _node_body(h_ref, p_ref, wh_ref, wp_ref, b_ref, a_ref, bn_ref):
    pw = p_ref[...] @ wp_ref[...]
    a_ref[...] = h_ref[...] @ wh_ref[...] + pw + b_ref[...]
    bn_ref[...] = -pw


def _node_precompute(h, pos16, Wh_part, Wp16, b):
    d = h.shape[1]
    return pl.pallas_call(
        _node_body,
        grid=(N // NBLK,),
        in_specs=[
            pl.BlockSpec((NBLK, d), lambda i: (i, 0)),
            pl.BlockSpec((NBLK, 16), lambda i: (i, 0)),
            pl.BlockSpec((d, H), lambda i: (0, 0)),
            pl.BlockSpec((16, H), lambda i: (0, 0)),
            pl.BlockSpec((1, H), lambda i: (0, 0)),
        ],
        out_specs=[pl.BlockSpec((NBLK, H), lambda i: (i, 0)),
                   pl.BlockSpec((NBLK, H), lambda i: (i, 0))],
        out_shape=[jax.ShapeDtypeStruct((N, H), jnp.float32),
                   jax.ShapeDtypeStruct((N, H), jnp.float32)],
    )(h, pos16, Wh_part, Wp16, b.reshape(1, H))


def _mlp_body(z_ref, w2_ref, b2_ref, m_ref):
    z = z_ref[...].reshape(NBLK, 2 * H)
    a1 = jnp.maximum(z, 0.0)
    m_ref[...] = jnp.maximum(a1 @ w2_ref[...] + b2_ref[...], 0.0)


def _edge_mlp(Z1, W2, b2):
    return pl.pallas_call(
        _mlp_body,
        grid=(EROW // NBLK,),
        in_specs=[
            pl.BlockSpec((NBLK * 2 * H,), lambda i: (i,)),
            pl.BlockSpec((2 * H, 2 * H), lambda i: (0, 0)),
            pl.BlockSpec((1, 2 * H), lambda i: (0, 0)),
        ],
        out_specs=pl.BlockSpec((NBLK, 2 * H), lambda i: (i, 0)),
        out_shape=jax.ShapeDtypeStruct((EROWP, 2 * H), jnp.float32),
    )(Z1, W2, b2)


def _head_body(h_ref, w_ref, b_ref, o_ref):
    o_ref[...] = h_ref[...] @ w_ref[...] + b_ref[...]


def _head(h, Wh, bh):
    return pl.pallas_call(
        _head_body,
        grid=(N // NBLK,),
        in_specs=[
            pl.BlockSpec((NBLK, H), lambda i: (i, 0)),
            pl.BlockSpec((H, OUT), lambda i: (0, 0)),
            pl.BlockSpec((1, OUT), lambda i: (0, 0)),
        ],
        out_specs=pl.BlockSpec((NBLK, OUT), lambda i: (i, 0)),
        out_shape=jax.ShapeDtypeStruct((N, OUT), jnp.float32),
    )(h, Wh, bh.reshape(1, OUT))


_ZCH = (KG // 2) * 2 * H


@functools.partial(
    pl.kernel,
    out_type=jax.ShapeDtypeStruct((EROWP * 2 * H,), jnp.float32),
    mesh=_MESH,
    scratch_types=[
        pltpu.VMEM((NBUF, 128), jnp.int32),
        pltpu.VMEM((NBUF, 128), jnp.int32),
        pltpu.VMEM((NBUF * 128,), jnp.int32),
        pltpu.VMEM((NBUF * 128,), jnp.int32),
        pltpu.VMEM((NBUF, KG, 2 * H), jnp.float32),
        pltpu.VMEM((NBUF, KG, 2 * H), jnp.float32),
        pltpu.VMEM((NBUF * _ZCH,), jnp.float32),
        (pltpu.SemaphoreType.DMA, pltpu.SemaphoreType.DMA),
        (pltpu.SemaphoreType.DMA, pltpu.SemaphoreType.DMA),
        (pltpu.SemaphoreType.DMA, pltpu.SemaphoreType.DMA),
        (pltpu.SemaphoreType.DMA, pltpu.SemaphoreType.DMA),
    ],
)
def _gather_sc(sh2_hbm, dh2_hbm, ss2_hbm, sd2_hbm, a_hbm, bn_hbm, z_hbm,
               idx_s, idx_d, raw_s, raw_d, buf_a, buf_b, zbuf,
               sem_i, sem_g, sem_z, sem_r):
    w = lax.axis_index("s") * 2 + lax.axis_index("c")
    nch = (NCHG - w + NW - 1) // NW

    def cid(i):
        return w + i * NW

    def issue_idx(i, b):
        c = cid(i)
        pltpu.async_copy(sh2_hbm.at[c], idx_s.at[b], sem_i[b])
        pltpu.async_copy(dh2_hbm.at[c], idx_d.at[b], sem_i[b])

    def wait_idx(i, b):
        c = cid(i)
        pltpu.make_async_copy(sh2_hbm.at[c], idx_s.at[b], sem_i[b]).wait()
        pltpu.make_async_copy(dh2_hbm.at[c], idx_d.at[b], sem_i[b]).wait()

    def issue_raw(i, b):
        c = cid(i)
        pltpu.async_copy(ss2_hbm.at[c], raw_s.at[pl.ds(b * 128, 128)],
                         sem_r[b])
        pltpu.async_copy(sd2_hbm.at[c], raw_d.at[pl.ds(b * 128, 128)],
                         sem_r[b])

    def wait_raw(i, b):
        c = cid(i)
        pltpu.make_async_copy(ss2_hbm.at[c], raw_s.at[pl.ds(b * 128, 128)],
                              sem_r[b]).wait()
        pltpu.make_async_copy(sd2_hbm.at[c], raw_d.at[pl.ds(b * 128, 128)],
                              sem_r[b]).wait()

    _GW = 32

    def issue_gather(b):
        for s in range(KG // _GW):
            pltpu.async_copy(a_hbm.at[idx_s.at[b, pl.ds(s * _GW, _GW)]],
                             buf_a.at[b, pl.ds(s * _GW, _GW), :], sem_g[b])
            pltpu.async_copy(bn_hbm.at[idx_d.at[b, pl.ds(s * _GW, _GW)]],
                             buf_b.at[b, pl.ds(s * _GW, _GW), :], sem_g[b])

    def wait_gather(b):
        for s in range(KG // _GW):
            pltpu.make_async_copy(
                a_hbm.at[idx_s.at[b, pl.ds(s * _GW, _GW)]],
                buf_a.at[b, pl.ds(s * _GW, _GW), :], sem_g[b]).wait()
            pltpu.make_async_copy(
                bn_hbm.at[idx_d.at[b, pl.ds(s * _GW, _GW)]],
                buf_b.at[b, pl.ds(s * _GW, _GW), :], sem_g[b]).wait()

    def issue_z(i, b):
        c = cid(i)
        pltpu.async_copy(zbuf.at[pl.ds(b * _ZCH, _ZCH)],
                         z_hbm.at[pl.ds(c * _ZCH, _ZCH)], sem_z[b])

    def wait_z(i, b):
        c = cid(i)
        pltpu.make_async_copy(zbuf.at[pl.ds(b * _ZCH, _ZCH)],
                              z_hbm.at[pl.ds(c * _ZCH, _ZCH)],
                              sem_z[b]).wait()

    def assemble(b):
        for g in range(KG // 16):
            svec = lax.rem(raw_s[pl.ds(b * 128 + g * 16, 16)], 2) * 64
            dvec = lax.rem(raw_d[pl.ds(b * 128 + g * 16, 16)], 2) * 64
            zb = b * _ZCH + g * 16 * 64
            for e16 in range(16):
                e = g * 16 + e16
                soff = svec[e16]
                doff = dvec[e16]
                for j in range(4):
                    av = buf_a[b, e, pl.ds(soff + j * 16, 16)]
                    bv = buf_b[b, e, pl.ds(doff + j * 16, 16)]
                    zbuf[pl.ds(zb + e16 * 64 + j * 16, 16)] = av + bv

    @pl.when(nch > 0)
    def _():
        issue_idx(0, 0)
        issue_raw(0, 0)

    @pl.when(nch > 1)
    def _():
        issue_idx(1, 1)
        issue_raw(1, 1)

    @pl.when(nch > 0)
    def _():
        wait_idx(0, 0)
        issue_gather(0)

    def body(i2, carry):
        c0 = 2 * i2
        c1 = c0 + 1

        @pl.when(c0 < nch)
        def _():
            wait_gather(0)

        @pl.when(c0 + 2 < nch)
        def _():
            issue_idx(c0 + 2, 0)

        @pl.when(c1 < nch)
        def _():
            wait_idx(c1, 1)
            issue_gather(1)

        @pl.when(c0 < nch)
        def _():
            @pl.when(c0 >= 2)
            def _():
                wait_z(c0 - 2, 0)

            wait_raw(c0, 0)
            assemble(0)
            issue_z(c0, 0)

        @pl.when(c0 + 2 < nch)
        def _():
            issue_raw(c0 + 2, 0)

        @pl.when(c1 < nch)
        def _():
            wait_gather(1)

        @pl.when(c1 + 2 < nch)
        def _():
            issue_idx(c1 + 2, 1)

        @pl.when(c0 + 2 < nch)
        def _():
            wait_idx(c0 + 2, 0)
            issue_gather(0)

        @pl.when(c1 < nch)
        def _():
            @pl.when(c1 >= 2)
            def _():
                wait_z(c1 - 2, 1)

            wait_raw(c1, 1)
            assemble(1)
            issue_z(c1, 1)

        @pl.when(c1 + 2 < nch)
        def _():
            issue_raw(c1 + 2, 1)

        return carry

    lax.fori_loop(0, (nch + 1) // 2, body, 0)

    for b in range(NBUF):
        for back in (1, 2):
            @pl.when((nch >= back) & (lax.rem(nch - back, NBUF) == b))
            def _(back=back, b=b):
                wait_z(nch - back, b)


@functools.partial(
    pl.kernel,
    out_type=jax.ShapeDtypeStruct((NPAD * H,), jnp.float32),
    mesh=_MESH,
    scratch_types=[
        pltpu.VMEM((8, 128), jnp.int32),
        pltpu.VMEM((NBUF, KS), jnp.int32),
        pltpu.VMEM((NBUF, KS // 2, 2 * H), jnp.float32),
        pltpu.VMEM((RNG * H,), jnp.float32),
        (pltpu.SemaphoreType.DMA, pltpu.SemaphoreType.DMA),
    ],
)
def _scatter_sc(sd_hbm, m_hbm, bnd_hbm, zero_hbm, out_hbm,
                bnd_v, sd_v, m_v, tbl, sem):
    w = lax.axis_index("s") * 2 + lax.axis_index("c")
    pltpu.sync_copy(bnd_hbm.at[w], bnd_v)
    pltpu.sync_copy(zero_hbm, tbl)

    bv = bnd_v[0, pl.ds(0, 16)]
    lo = bv[0]
    hi = bv[1]
    start_node = pl.multiple_of(w * RNG, 8)
    abase = pl.multiple_of((lo // 16) * 16, 16)
    nch = (hi - abase + KS - 1) // KS

    def issue(c, buf):
        b = pl.multiple_of(abase + c * KS, 16)
        pltpu.async_copy(sd_hbm.at[pl.ds(b, KS)], sd_v.at[buf], sem[buf])
        pltpu.async_copy(m_hbm.at[pl.ds(pl.multiple_of(b // 2, 8), KS // 2),
                                  :], m_v.at[buf], sem[buf])

    def await_chunk(c, buf):
        b = pl.multiple_of(abase + c * KS, 16)
        pltpu.make_async_copy(sd_hbm.at[pl.ds(b, KS)], sd_v.at[buf],
                              sem[buf]).wait()
        pltpu.make_async_copy(m_hbm.at[pl.ds(pl.multiple_of(b // 2, 8),
                                             KS // 2), :],
                              m_v.at[buf], sem[buf]).wait()

    @pl.when(nch > 0)
    def _():
        issue(0, 0)

    @pl.when(nch > 1)
    def _():
        issue(1, 1)

    zeros = jnp.zeros((16,), jnp.float32)

    def process(c, buf, carry):
        prev_row, r0, r1, r2, r3 = carry
        b = abase + c * KS
        for g in range(KS // 16):
            dv = sd_v[buf, pl.ds(g * 16, 16)]
            rows = dv - start_node
            for e16 in range(16):
                ge = b + g * 16 + e16
                valid = (ge >= lo) & (ge < hi)
                row = rows[e16]
                enew = valid & (row != prev_row)
                mrow = g * 8 + e16 // 2
                moff = (e16 % 2) * 64
                m0 = m_v[buf, mrow, pl.ds(moff, 16)]
                m1 = m_v[buf, mrow, pl.ds(moff + 16, 16)]
                m2 = m_v[buf, mrow, pl.ds(moff + 32, 16)]
                m3 = m_v[buf, mrow, pl.ds(moff + 48, 16)]

                @pl.when(enew & (prev_row >= 0))
                def _(pr=prev_row, s0=r0, s1=r1, s2=r2, s3=r3):
                    tb = pr * H
                    tbl[pl.ds(tb, 16)] = s0
                    tbl[pl.ds(tb + 16, 16)] = s1
                    tbl[pl.ds(tb + 32, 16)] = s2
                    tbl[pl.ds(tb + 48, 16)] = s3

                r0 = jnp.where(valid,
                               jnp.where(enew, m0, jnp.maximum(r0, m0)), r0)
                r1 = jnp.where(valid,
                               jnp.where(enew, m1, jnp.maximum(r1, m1)), r1)
                r2 = jnp.where(valid,
                               jnp.where(enew, m2, jnp.maximum(r2, m2)), r2)
                r3 = jnp.where(valid,
                               jnp.where(enew, m3, jnp.maximum(r3, m3)), r3)
                prev_row = jnp.where(enew, row, prev_row)
        return prev_row, r0, r1, r2, r3

    def body(i2, carry):
        for buf in range(NBUF):
            c = i2 * NBUF + buf

            @pl.when(c < nch)
            def _(c=c, buf=buf):
                await_chunk(c, buf)

            carry = process(c, buf, carry)

            @pl.when(c + NBUF < nch)
            def _(c=c, buf=buf):
                issue(c + NBUF, buf)
        return carry

    prev_row, r0, r1, r2, r3 = lax.fori_loop(
        0, (nch + NBUF - 1) // NBUF, body,
        (jnp.int32(-1), zeros, zeros, zeros, zeros))

    @pl.when(prev_row >= 0)
    def _():
        tb = prev_row * H
        tbl[pl.ds(tb, 16)] = r0
        tbl[pl.ds(tb + 16, 16)] = r1
        tbl[pl.ds(tb + 32, 16)] = r2
        tbl[pl.ds(tb + 48, 16)] = r3

    pltpu.sync_copy(tbl,
                    out_hbm.at[pl.ds(pl.multiple_of(start_node * H, 128),
                                     RNG * H)])


_XLA_SCATTER = False
_XLA_GATHER = False


def kernel(x, pos, edge_index, W0a, b0a, W0b, b0b, W1a, b1a, W1b, b1b,
           W2a, b2a, W2b, b2b, Wh, bh):
    src = edge_index[0]
    dst = edge_index[1]
    sd, ss = lax.sort([dst, src], num_keys=1)
    node_starts = jnp.arange(NW + 1, dtype=jnp.int32) * RNG
    bnd = jnp.searchsorted(sd, node_starts, side="left").astype(jnp.int32)
    bnd3 = (jnp.zeros((NW, 8, 128), jnp.int32)
            .at[:, 0, 0].set(bnd[:NW]).at[:, 0, 1].set(bnd[1:NW + 1]))
    sdp = jnp.pad(sd, (0, KS))
    ss2 = ss.reshape(E // 128, 128)
    sd2 = sd.reshape(E // 128, 128)
    sh2 = ss2 // 2
    dh2 = sd2 // 2
    pos16 = jnp.zeros((N, 16), jnp.float32).at[:, :3].set(pos)
    zero_tbl = jnp.zeros((RNG * H,), jnp.float32)

    h = x
    for Wa, ba, Wb, bb in ((W0a, b0a, W0b, b0b), (W1a, b1a, W1b, b1b),
                           (W2a, b2a, W2b, b2b)):
        d = h.shape[1]
        Wp16 = jnp.zeros((16, H), jnp.float32).at[:3].set(Wa[d:d + 3])
        A, Bn = _node_precompute(h, pos16, Wa[:d], Wp16, ba)
        if _XLA_GATHER:
            z = A[ss] + Bn[sd]
            Z2 = jnp.zeros((EROWP * 2 * H,), jnp.float32).at[:E * H].set(
                z.reshape(-1))
        else:
            Z2 = _gather_sc(sh2, dh2, ss2, sd2, A.reshape(N // 2, 2 * H),
                            Bn.reshape(N // 2, 2 * H))
        W2 = jnp.block([[Wb, jnp.zeros((H, H), jnp.float32)],
                        [jnp.zeros((H, H), jnp.float32), Wb]])
        b2 = jnp.concatenate([bb, bb]).reshape(1, 2 * H)
        M2 = _edge_mlp(Z2, W2, b2)
        if _XLA_SCATTER:
            m = M2[:E // 2].reshape(E, H)
            h = jax.ops.segment_max(m, sd, num_segments=N)
            h = jnp.where(jnp.isneginf(h), 0.0, h)
        else:
            hp = _scatter_sc(sdp, M2, bnd3, zero_tbl)
            h = hp.reshape(NPAD, H)[:N]
    return _head(h, Wh, bh)


# speedup vs baseline: 2.6514x; 1.0001x over previous
import functools

import jax
import jax.numpy as jnp
from jax import lax
from jax.experimental import pallas as pl
from jax.experimental.pallas import tpu as pltpu
from jax.experimental.pallas import tpu_sc as plsc

N = 50000
E = 800000
H = 64
OUT = 2

NW = 32
RNG = 1568
NPAD = NW * RNG
KG = 128
NCHG = E // KG
KS = 128
NBUF = 2
EROW = E // 2
EROWP = (E + KS) // 2
NBLK = 2000

_MESH = plsc.VectorSubcoreMesh(
    core_axis_name="c", subcore_axis_name="s", num_cores=2, num_subcores=16)


def _node_body(h_ref, p_ref, wh_ref, wp_ref, b_ref, a_ref, bn_ref):
    pw = p_ref[...] @ wp_ref[...]
    a_ref[...] = h_ref[...] @ wh_ref[...] + pw + b_ref[...]
    bn_ref[...] = -pw


def _node_precompute(h, pos16, Wh_part, Wp16, b):
    d = h.shape[1]
    return pl.pallas_call(
        _node_body,
        grid=(N // NBLK,),
        in_specs=[
            pl.BlockSpec((NBLK, d), lambda i: (i, 0)),
            pl.BlockSpec((NBLK, 16), lambda i: (i, 0)),
            pl.BlockSpec((d, H), lambda i: (0, 0)),
            pl.BlockSpec((16, H), lambda i: (0, 0)),
            pl.BlockSpec((1, H), lambda i: (0, 0)),
        ],
        out_specs=[pl.BlockSpec((NBLK, H), lambda i: (i, 0)),
                   pl.BlockSpec((NBLK, H), lambda i: (i, 0))],
        out_shape=[jax.ShapeDtypeStruct((N, H), jnp.float32),
                   jax.ShapeDtypeStruct((N, H), jnp.float32)],
    )(h, pos16, Wh_part, Wp16, b.reshape(1, H))


def _mlp_body(z_ref, w2_ref, b2_ref, m_ref):
    z = z_ref[...].reshape(NBLK, 2 * H)
    a1 = jnp.maximum(z, 0.0)
    m_ref[...] = jnp.maximum(a1 @ w2_ref[...] + b2_ref[...], 0.0)


def _edge_mlp(Z1, W2, b2):
    return pl.pallas_call(
        _mlp_body,
        grid=(EROW // NBLK,),
        in_specs=[
            pl.BlockSpec((NBLK * 2 * H,), lambda i: (i,)),
            pl.BlockSpec((2 * H, 2 * H), lambda i: (0, 0)),
            pl.BlockSpec((1, 2 * H), lambda i: (0, 0)),
        ],
        out_specs=pl.BlockSpec((NBLK, 2 * H), lambda i: (i, 0)),
        out_shape=jax.ShapeDtypeStruct((EROWP, 2 * H), jnp.float32),
    )(Z1, W2, b2)


def _head_body(h_ref, w_ref, b_ref, o_ref):
    o_ref[...] = h_ref[...] @ w_ref[...] + b_ref[...]


def _head(h, Wh, bh):
    return pl.pallas_call(
        _head_body,
        grid=(N // NBLK,),
        in_specs=[
            pl.BlockSpec((NBLK, H), lambda i: (i, 0)),
            pl.BlockSpec((H, OUT), lambda i: (0, 0)),
            pl.BlockSpec((1, OUT), lambda i: (0, 0)),
        ],
        out_specs=pl.BlockSpec((NBLK, OUT), lambda i: (i, 0)),
        out_shape=jax.ShapeDtypeStruct((N, OUT), jnp.float32),
    )(h, Wh, bh.reshape(1, OUT))


_ZCH = (KG // 2) * 2 * H


@functools.partial(
    pl.kernel,
    out_type=jax.ShapeDtypeStruct((EROWP * 2 * H,), jnp.float32),
    mesh=_MESH,
    scratch_types=[
        pltpu.VMEM((NBUF, 128), jnp.int32),
        pltpu.VMEM((NBUF, 128), jnp.int32),
        pltpu.VMEM((NBUF * 128,), jnp.int32),
        pltpu.VMEM((NBUF * 128,), jnp.int32),
        pltpu.VMEM((NBUF, KG, 2 * H), jnp.float32),
        pltpu.VMEM((NBUF, KG, 2 * H), jnp.float32),
        pltpu.VMEM((NBUF * _ZCH,), jnp.float32),
        (pltpu.SemaphoreType.DMA, pltpu.SemaphoreType.DMA),
        (pltpu.SemaphoreType.DMA, pltpu.SemaphoreType.DMA),
        (pltpu.SemaphoreType.DMA, pltpu.SemaphoreType.DMA),
        (pltpu.SemaphoreType.DMA, pltpu.SemaphoreType.DMA),
    ],
)
def _gather_sc(sh2_hbm, dh2_hbm, ss2_hbm, sd2_hbm, a_hbm, bn_hbm, z_hbm,
               idx_s, idx_d, raw_s, raw_d, buf_a, buf_b, zbuf,
               sem_i, sem_g, sem_z, sem_r):
    w = lax.axis_index("s") * 2 + lax.axis_index("c")
    nch = (NCHG - w + NW - 1) // NW

    def cid(i):
        return w + i * NW

    def issue_idx(i, b):
        c = cid(i)
        pltpu.async_copy(sh2_hbm.at[c], idx_s.at[b], sem_i[b])
        pltpu.async_copy(dh2_hbm.at[c], idx_d.at[b], sem_i[b])

    def wait_idx(i, b):
        c = cid(i)
        pltpu.make_async_copy(sh2_hbm.at[c], idx_s.at[b], sem_i[b]).wait()
        pltpu.make_async_copy(dh2_hbm.at[c], idx_d.at[b], sem_i[b]).wait()

    def issue_raw(i, b):
        c = cid(i)
        pltpu.async_copy(ss2_hbm.at[c], raw_s.at[pl.ds(b * 128, 128)],
                         sem_r[b])
        pltpu.async_copy(sd2_hbm.at[c], raw_d.at[pl.ds(b * 128, 128)],
                         sem_r[b])

    def wait_raw(i, b):
        c = cid(i)
        pltpu.make_async_copy(ss2_hbm.at[c], raw_s.at[pl.ds(b * 128, 128)],
                              sem_r[b]).wait()
        pltpu.make_async_copy(sd2_hbm.at[c], raw_d.at[pl.ds(b * 128, 128)],
                              sem_r[b]).wait()

    _GW = 32

    def issue_gather(b):
        for s in range(KG // _GW):
            pltpu.async_copy(a_hbm.at[idx_s.at[b, pl.ds(s * _GW, _GW)]],
                             buf_a.at[b, pl.ds(s * _GW, _GW), :], sem_g[b])
            pltpu.async_copy(bn_hbm.at[idx_d.at[b, pl.ds(s * _GW, _GW)]],
                             buf_b.at[b, pl.ds(s * _GW, _GW), :], sem_g[b])

    def wait_gather(b):
        for s in range(KG // _GW):
            pltpu.make_async_copy(
                a_hbm.at[idx_s.at[b, pl.ds(s * _GW, _GW)]],
                buf_a.at[b, pl.ds(s * _GW, _GW), :], sem_g[b]).wait()
            pltpu.make_async_copy(
                bn_hbm.at[idx_d.at[b, pl.ds(s * _GW, _GW)]],
                buf_b.at[b, pl.ds(s * _GW, _GW), :], sem_g[b]).wait()

    def issue_z(i, b):
        c = cid(i)
        pltpu.async_copy(zbuf.at[pl.ds(b * _ZCH, _ZCH)],
                         z_hbm.at[pl.ds(c * _ZCH, _ZCH)], sem_z[b])

    def wait_z(i, b):
        c = cid(i)
        pltpu.make_async_copy(zbuf.at[pl.ds(b * _ZCH, _ZCH)],
                              z_hbm.at[pl.ds(c * _ZCH, _ZCH)],
                              sem_z[b]).wait()

    def assemble(b):
        for g in range(KG // 16):
            svec = lax.rem(raw_s[pl.ds(b * 128 + g * 16, 16)], 2) * 64
            dvec = lax.rem(raw_d[pl.ds(b * 128 + g * 16, 16)], 2) * 64
            zb = b * _ZCH + g * 16 * 64
            for e16 in range(16):
                e = g * 16 + e16
                soff = svec[e16]
                doff = dvec[e16]
                for j in range(4):
                    av = buf_a[b, e, pl.ds(soff + j * 16, 16)]
                    bv = buf_b[b, e, pl.ds(doff + j * 16, 16)]
                    zbuf[pl.ds(zb + e16 * 64 + j * 16, 16)] = av + bv

    @pl.when(nch > 0)
    def _():
        issue_idx(0, 0)
        issue_raw(0, 0)

    @pl.when(nch > 1)
    def _():
        issue_idx(1, 1)
        issue_raw(1, 1)

    @pl.when(nch > 0)
    def _():
        wait_idx(0, 0)
        issue_gather(0)

    def body(i2, carry):
        c0 = 2 * i2
        c1 = c0 + 1

        @pl.when(c0 < nch)
        def _():
            wait_gather(0)

        @pl.when(c0 + 2 < nch)
        def _():
            issue_idx(c0 + 2, 0)

        @pl.when(c1 < nch)
        def _():
            wait_idx(c1, 1)
            issue_gather(1)

        @pl.when(c0 < nch)
        def _():
            @pl.when(c0 >= 2)
            def _():
                wait_z(c0 - 2, 0)

            wait_raw(c0, 0)
            assemble(0)
            issue_z(c0, 0)

        @pl.when(c0 + 2 < nch)
        def _():
            issue_raw(c0 + 2, 0)

        @pl.when(c1 < nch)
        def _():
            wait_gather(1)

        @pl.when(c1 + 2 < nch)
        def _():
            issue_idx(c1 + 2, 1)

        @pl.when(c0 + 2 < nch)
        def _():
            wait_idx(c0 + 2, 0)
            issue_gather(0)

        @pl.when(c1 < nch)
        def _():
            @pl.when(c1 >= 2)
            def _():
                wait_z(c1 - 2, 1)

            wait_raw(c1, 1)
            assemble(1)
            issue_z(c1, 1)

        @pl.when(c1 + 2 < nch)
        def _():
            issue_raw(c1 + 2, 1)

        return carry

    lax.fori_loop(0, (nch + 1) // 2, body, 0)

    for b in range(NBUF):
        for back in (1, 2):
            @pl.when((nch >= back) & (lax.rem(nch - back, NBUF) == b))
            def _(back=back, b=b):
                wait_z(nch - back, b)


@functools.partial(
    pl.kernel,
    out_type=jax.ShapeDtypeStruct((NPAD * H,), jnp.float32),
    mesh=_MESH,
    scratch_types=[
        pltpu.VMEM((8, 128), jnp.int32),
        pltpu.VMEM((NBUF, KS), jnp.int32),
        pltpu.VMEM((NBUF, KS // 2, 2 * H), jnp.float32),
        pltpu.VMEM((RNG * H,), jnp.float32),
        (pltpu.SemaphoreType.DMA, pltpu.SemaphoreType.DMA),
    ],
)
def _scatter_sc(sd_hbm, m_hbm, bnd_hbm, zero_hbm, out_hbm,
                bnd_v, sd_v, m_v, tbl, sem):
    w = lax.axis_index("s") * 2 + lax.axis_index("c")
    pltpu.sync_copy(bnd_hbm.at[w], bnd_v)
    pltpu.sync_copy(zero_hbm, tbl)

    bv = bnd_v[0, pl.ds(0, 16)]
    lo = bv[0]
    hi = bv[1]
    start_node = pl.multiple_of(w * RNG, 8)
    abase = pl.multiple_of((lo // 16) * 16, 16)
    nch = (hi - abase + KS - 1) // KS

    def issue(c, buf):
        b = pl.multiple_of(abase + c * KS, 16)
        pltpu.async_copy(sd_hbm.at[pl.ds(b, KS)], sd_v.at[buf], sem[buf])
        pltpu.async_copy(m_hbm.at[pl.ds(pl.multiple_of(b // 2, 8), KS // 2),
                                  :], m_v.at[buf], sem[buf])

    def await_chunk(c, buf):
        b = pl.multiple_of(abase + c * KS, 16)
        pltpu.make_async_copy(sd_hbm.at[pl.ds(b, KS)], sd_v.at[buf],
                              sem[buf]).wait()
        pltpu.make_async_copy(m_hbm.at[pl.ds(pl.multiple_of(b // 2, 8),
                                             KS // 2), :],
                              m_v.at[buf], sem[buf]).wait()

    @pl.when(nch > 0)
    def _():
        issue(0, 0)

    @pl.when(nch > 1)
    def _():
        issue(1, 1)

    zeros = jnp.zeros((16,), jnp.float32)

    def process(c, buf, carry):
        prev_row, r0, r1, r2, r3 = carry
        b = abase + c * KS
        for g in range(KS // 16):
            dv = sd_v[buf, pl.ds(g * 16, 16)]
            rows = dv - start_node
            for e16 in range(16):
                ge = b + g * 16 + e16
                valid = (ge >= lo) & (ge < hi)
                row = rows[e16]
                enew = valid & (row != prev_row)
                mrow = g * 8 + e16 // 2
                moff = (e16 % 2) * 64
                m0 = m_v[buf, mrow, pl.ds(moff, 16)]
                m1 = m_v[buf, mrow, pl.ds(moff + 16, 16)]
                m2 = m_v[buf, mrow, pl.ds(moff + 32, 16)]
                m3 = m_v[buf, mrow, pl.ds(moff + 48, 16)]

                @pl.when(enew & (prev_row >= 0))
                def _(pr=prev_row, s0=r0, s1=r1, s2=r2, s3=r3):
                    tb = pr * H
                    tbl[pl.ds(tb, 16)] = s0
                    tbl[pl.ds(tb + 16, 16)] = s1
                    tbl[pl.ds(tb + 32, 16)] = s2
                    tbl[pl.ds(tb + 48, 16)] = s3

                r0 = jnp.where(valid,
                               jnp.where(enew, m0, jnp.maximum(r0, m0)), r0)
                r1 = jnp.where(valid,
                               jnp.where(enew, m1, jnp.maximum(r1, m1)), r1)
                r2 = jnp.where(valid,
                               jnp.where(enew, m2, jnp.maximum(r2, m2)), r2)
                r3 = jnp.where(valid,
                               jnp.where(enew, m3, jnp.maximum(r3, m3)), r3)
                prev_row = jnp.where(enew, row, prev_row)
        return prev_row, r0, r1, r2, r3

    def body(i2, carry):
        for buf in range(NBUF):
            c = i2 * NBUF + buf

            @pl.when(c < nch)
            def _(c=c, buf=buf):
                await_chunk(c, buf)

            carry = process(c, buf, carry)

            @pl.when(c + NBUF < nch)
            def _(c=c, buf=buf):
                issue(c + NBUF, buf)
        return carry

    prev_row, r0, r1, r2, r3 = lax.fori_loop(
        0, (nch + NBUF - 1) // NBUF, body,
        (jnp.int32(-1), zeros, zeros, zeros, zeros))

    @pl.when(prev_row >= 0)
    def _():
        tb = prev_row * H
        tbl[pl.ds(tb, 16)] = r0
        tbl[pl.ds(tb + 16, 16)] = r1
        tbl[pl.ds(tb + 32, 16)] = r2
        tbl[pl.ds(tb + 48, 16)] = r3

    pltpu.sync_copy(tbl,
                    out_hbm.at[pl.ds(pl.multiple_of(start_node * H, 128),
                                     RNG * H)])


def kernel(x, pos, edge_index, W0a, b0a, W0b, b0b, W1a, b1a, W1b, b1b,
           W2a, b2a, W2b, b2b, Wh, bh):
    src = edge_index[0]
    dst = edge_index[1]
    sd, ss = lax.sort([dst, src], num_keys=1)
    node_starts = jnp.arange(NW + 1, dtype=jnp.int32) * RNG
    bnd = jnp.searchsorted(sd, node_starts, side="left").astype(jnp.int32)
    bnd3 = (jnp.zeros((NW, 8, 128), jnp.int32)
            .at[:, 0, 0].set(bnd[:NW]).at[:, 0, 1].set(bnd[1:NW + 1]))
    sdp = jnp.pad(sd, (0, KS))
    ss2 = ss.reshape(E // 128, 128)
    sd2 = sd.reshape(E // 128, 128)
    sh2 = ss2 // 2
    dh2 = sd2 // 2
    pos16 = jnp.zeros((N, 16), jnp.float32).at[:, :3].set(pos)
    zero_tbl = jnp.zeros((RNG * H,), jnp.float32)

    h = x
    for Wa, ba, Wb, bb in ((W0a, b0a, W0b, b0b), (W1a, b1a, W1b, b1b),
                           (W2a, b2a, W2b, b2b)):
        d = h.shape[1]
        Wp16 = jnp.zeros((16, H), jnp.float32).at[:3].set(Wa[d:d + 3])
        A, Bn = _node_precompute(h, pos16, Wa[:d], Wp16, ba)
        Z2 = _gather_sc(sh2, dh2, ss2, sd2, A.reshape(N // 2, 2 * H),
                        Bn.reshape(N // 2, 2 * H))
        W2 = jnp.block([[Wb, jnp.zeros((H, H), jnp.float32)],
                        [jnp.zeros((H, H), jnp.float32), Wb]])
        b2 = jnp.concatenate([bb, bb]).reshape(1, 2 * H)
        M2 = _edge_mlp(Z2, W2, b2)
        hp = _scatter_sc(sdp, M2, bnd3, zero_tbl)
        h = hp.reshape(NPAD, H)[:N]
    return _head(h, Wh, bh)
